# Initial kernel scaffold; baseline (speedup 1.0000x reference)
import jax
import jax.numpy as jnp
from jax.experimental import pallas as pl


def kernel(logits, default_boxes):
    raise NotImplementedError("write your pallas kernel here")


# speedup vs baseline: 17.3804x; 17.3804x over previous
import functools

import jax
import jax.numpy as jnp
from jax.experimental import pallas as pl
from jax.experimental.pallas import tpu as pltpu

MAX_OUT = 100
IOU_THR = 0.5
SCORE_THR = 0.01
ROWS = 160
LANES = 128
NPAD = ROWS * LANES
NEG_INF = float("-inf")


def _decode_nms_kernel(lt_ref, db_ref, boxes_ref, cls_ref, sc_ref, num_ref, s_scr):
    ax1 = db_ref[0]
    ay1 = db_ref[1]
    ax2 = db_ref[2]
    ay2 = db_ref[3]
    acx = (ax2 + ax1) * 0.5
    acy = (ay2 + ay1) * 0.5
    aw = ax2 - ax1
    ah = ay2 - ay1

    pcx = lt_ref[0, 0] * aw + acx
    pcy = lt_ref[0, 1] * ah + acy
    pw = jnp.exp(lt_ref[0, 2]) * aw
    ph = jnp.exp(lt_ref[0, 3]) * ah
    x1 = jnp.clip(pcx - pw * 0.5, 0.0, 1.0)
    y1 = jnp.clip(pcy - ph * 0.5, 0.0, 1.0)
    x2 = jnp.clip(pcx + pw * 0.5, 0.0, 1.0)
    y2 = jnp.clip(pcy + ph * 0.5, 0.0, 1.0)
    areas = (x2 - x1) * (y2 - y1)

    m = lt_ref[0, 4]
    for c in range(5, 25):
        m = jnp.maximum(m, lt_ref[0, c])
    ssum = jnp.exp(lt_ref[0, 4] - m)
    best = lt_ref[0, 4]
    cls = jnp.zeros((ROWS, LANES), dtype=jnp.int32)
    for c in range(5, 25):
        lc = lt_ref[0, c]
        ssum = ssum + jnp.exp(lc - m)
        gt = lc > best
        best = jnp.where(gt, lc, best)
        cls = jnp.where(gt, jnp.int32(c - 4), cls)
    score = 1.0 / ssum

    s0 = jnp.where(cls != 0, score, NEG_INF)
    s0 = jnp.where(s0 < SCORE_THR, NEG_INF, s0)
    s_scr[:, :] = s0

    iota = (
        jax.lax.broadcasted_iota(jnp.int32, (ROWS, LANES), 0) * LANES
        + jax.lax.broadcasted_iota(jnp.int32, (ROWS, LANES), 1)
    )

    boxes_ref[...] = jnp.zeros_like(boxes_ref)
    cls_ref[...] = jnp.zeros_like(cls_ref)
    sc_ref[...] = jnp.zeros_like(sc_ref)

    def body(t, n):
        s = s_scr[:, :]
        msc = jnp.max(s)
        valid = msc != NEG_INF
        idx = jnp.min(jnp.where(s == msc, iota, jnp.int32(NPAD)))
        sel = iota == idx
        bx1 = jnp.sum(jnp.where(sel, x1, 0.0))
        by1 = jnp.sum(jnp.where(sel, y1, 0.0))
        bx2 = jnp.sum(jnp.where(sel, x2, 0.0))
        by2 = jnp.sum(jnp.where(sel, y2, 0.0))
        barea = jnp.sum(jnp.where(sel, areas, 0.0))
        bsc = jnp.sum(jnp.where(sel, score, 0.0))
        bcls = jnp.sum(jnp.where(sel, cls, 0))

        xx1 = jnp.maximum(bx1, x1)
        yy1 = jnp.maximum(by1, y1)
        xx2 = jnp.minimum(bx2, x2)
        yy2 = jnp.minimum(by2, y2)
        inter = jnp.maximum(xx2 - xx1, 0.0) * jnp.maximum(yy2 - yy1, 0.0)
        iou = inter / (barea + areas - inter + 1e-9)
        supp = (iou > IOU_THR) | sel

        @pl.when(valid)
        def _():
            s_scr[:, :] = jnp.where(supp, NEG_INF, s)
            li = jax.lax.broadcasted_iota(jnp.int32, (1, 4), 1)
            row = jnp.where(
                li == 0, bx1, jnp.where(li == 1, by1, jnp.where(li == 2, bx2, by2))
            )
            boxes_ref[0, pl.ds(t, 1), :] = row
            cls_ref[0, pl.ds(t, 1), :] = jnp.reshape(bcls, (1, 1))
            sc_ref[0, pl.ds(t, 1), :] = jnp.reshape(bsc, (1, 1))

        return n + valid.astype(jnp.int32)

    n = jax.lax.fori_loop(0, MAX_OUT, body, jnp.int32(0))
    num_ref[0] = jnp.reshape(n, (1, 1))


@jax.jit
def _run(lt, db):
    return pl.pallas_call(
        _decode_nms_kernel,
        grid=(4,),
        in_specs=[
            pl.BlockSpec((1, 25, ROWS, LANES), lambda b: (b, 0, 0, 0)),
            pl.BlockSpec((4, ROWS, LANES), lambda b: (0, 0, 0)),
        ],
        out_specs=[
            pl.BlockSpec((1, MAX_OUT, 4), lambda b: (b, 0, 0)),
            pl.BlockSpec((1, MAX_OUT, 1), lambda b: (b, 0, 0)),
            pl.BlockSpec((1, MAX_OUT, 1), lambda b: (b, 0, 0)),
            pl.BlockSpec((1, 1, 1), lambda b: (b, 0, 0)),
        ],
        out_shape=[
            jax.ShapeDtypeStruct((4, MAX_OUT, 4), jnp.float32),
            jax.ShapeDtypeStruct((4, MAX_OUT, 1), jnp.int32),
            jax.ShapeDtypeStruct((4, MAX_OUT, 1), jnp.float32),
            jax.ShapeDtypeStruct((4, 1, 1), jnp.int32),
        ],
        scratch_shapes=[pltpu.VMEM((ROWS, LANES), jnp.float32)],
        compiler_params=pltpu.CompilerParams(
            dimension_semantics=("arbitrary",),
        ),
    )(lt, db)


def kernel(logits, default_boxes):
    b, n, c = logits.shape
    lt = jnp.transpose(logits, (0, 2, 1))
    lt = jnp.pad(lt, ((0, 0), (0, 0), (0, NPAD - n))).reshape(b, c, ROWS, LANES)
    db = jnp.pad(default_boxes.T, ((0, 0), (0, NPAD - n))).reshape(4, ROWS, LANES)
    det_boxes, det_cls, det_sc, det_num = _run(lt, db)
    return (
        det_boxes,
        det_cls[:, :, 0],
        det_sc[:, :, 0],
        det_num[:, 0, 0],
    )
